# baseline (speedup 1.0000x reference)
import functools

import jax
import jax.numpy as jnp
from jax import lax
from jax.experimental import pallas as pl
from jax.experimental.pallas import tpu as pltpu
from jax.experimental.pallas import tpu_sc as plsc

N = 10000
E = 160000
D = 256
H = 256
G = 64
L = 3
EPS = 1e-5
NEG = 0.01

NPD = 10240
BLK = 1280
NB = NPD // BLK

CH = 84
EP = 16 * CH * 128
STRIPE = NPD // 16


def _lrelu(t):
    return jnp.where(t >= 0, t, NEG * t)


def _edge_body(ztf, srci, dsti, out, src_v, dra, drb, bufa, bufb, acc,
               sema, semb, semda, semdb, sca, scb):
    c = lax.axis_index("c")
    s = lax.axis_index("s")
    pltpu.sync_copy(srci.at[c, s], src_v)

    z16 = jnp.zeros((16,), jnp.float32)

    def zrow(r, carry):
        for i in range(8):
            bufa[r, pl.ds(16 * i, 16)] = z16
        return carry
    lax.fori_loop(0, 128, zrow, 0)
    base = s * STRIPE
    for k in range(5):
        pltpu.sync_copy(bufa, acc.at[pl.ds(base + 128 * k, 128)])
    plsc.subcore_barrier()

    pltpu.async_copy(ztf.at[src_v.at[0]], bufa, sema)
    pltpu.async_copy(dsti.at[s, pl.ds(0, 1)], dra, semda)
    pltpu.async_copy(ztf.at[src_v.at[1]], bufb, semb)
    pltpu.async_copy(dsti.at[s, pl.ds(1, 1)], drb, semdb)

    def body(j, carry):
        pltpu.make_async_copy(ztf.at[src_v.at[2 * j]], bufa, sema).wait()
        pltpu.make_async_copy(dsti.at[s, pl.ds(2 * j, 1)], dra, semda).wait()
        pltpu.async_copy(bufa, acc.at[dra.at[0]], sca, add=True)
        pltpu.make_async_copy(ztf.at[src_v.at[2 * j + 1]], bufb, semb).wait()
        pltpu.make_async_copy(dsti.at[s, pl.ds(2 * j + 1, 1)], drb,
                              semdb).wait()
        pltpu.async_copy(bufb, acc.at[drb.at[0]], scb, add=True)
        pltpu.make_async_copy(bufa, acc.at[dra.at[0]], sca).wait()
        pltpu.async_copy(ztf.at[src_v.at[2 * j + 2]], bufa, sema)
        pltpu.async_copy(dsti.at[s, pl.ds(2 * j + 2, 1)], dra, semda)
        pltpu.make_async_copy(bufb, acc.at[drb.at[0]], scb).wait()
        pltpu.async_copy(ztf.at[src_v.at[2 * j + 3]], bufb, semb)
        pltpu.async_copy(dsti.at[s, pl.ds(2 * j + 3, 1)], drb, semdb)
        return carry
    lax.fori_loop(0, CH // 2, body, 0)
    pltpu.make_async_copy(ztf.at[src_v.at[CH]], bufa, sema).wait()
    pltpu.make_async_copy(dsti.at[s, pl.ds(CH, 1)], dra, semda).wait()
    pltpu.make_async_copy(ztf.at[src_v.at[CH + 1]], bufb, semb).wait()
    pltpu.make_async_copy(dsti.at[s, pl.ds(CH + 1, 1)], drb, semdb).wait()
    plsc.subcore_barrier()

    for k in range(5):
        sl = pl.ds(base + 128 * k, 128)
        pltpu.sync_copy(acc.at[sl], bufa)
        pltpu.sync_copy(bufa, out.at[c, sl])


def _edge_agg(ztf, srci, dsti):
    mesh = plsc.VectorSubcoreMesh(core_axis_name="c", subcore_axis_name="s",
                                  num_cores=2, num_subcores=16)
    return pl.kernel(
        _edge_body,
        out_type=jax.ShapeDtypeStruct((2, NPD, 128), jnp.float32),
        mesh=mesh,
        scratch_types=[
            pltpu.VMEM((CH + 2, 128), jnp.int32),
            pltpu.VMEM((1, 128), jnp.int32),
            pltpu.VMEM((1, 128), jnp.int32),
            pltpu.VMEM((128, 128), jnp.float32),
            pltpu.VMEM((128, 128), jnp.float32),
            pltpu.VMEM_SHARED((NPD, 128), jnp.float32),
            pltpu.SemaphoreType.DMA,
            pltpu.SemaphoreType.DMA,
            pltpu.SemaphoreType.DMA,
            pltpu.SemaphoreType.DMA,
            pltpu.SemaphoreType.DMA,
            pltpu.SemaphoreType.DMA,
        ],
    )(ztf, srci, dsti)


def _mmz_body(h_ref, dis_ref, w_ref, o_ref):
    ht = h_ref[...] * dis_ref[...]
    z = jnp.dot(ht, w_ref[...], preferred_element_type=jnp.float32)
    o_ref[0] = z[:, :128]
    o_ref[1] = z[:, 128:]


def _matmul_scaled(h, dis2d, w):
    return pl.pallas_call(
        _mmz_body,
        grid=(NB,),
        in_specs=[pl.BlockSpec((BLK, D), lambda i: (i, 0)),
                  pl.BlockSpec((BLK, 1), lambda i: (i, 0)),
                  pl.BlockSpec((D, H), lambda i: (0, 0))],
        out_specs=pl.BlockSpec((2, BLK, 128), lambda i: (0, i, 0)),
        out_shape=jax.ShapeDtypeStruct((2, NPD, 128), jnp.float32),
    )(h, dis2d, w)


def _stats_body(e_ref, dis_ref, oh_ref, b_ref, s_ref, q_ref):
    @pl.when(pl.program_id(0) == 0)
    def _():
        s_ref[...] = jnp.zeros_like(s_ref)
        q_ref[...] = jnp.zeros_like(q_ref)
    a = (jnp.concatenate([e_ref[0], e_ref[1]], axis=1) * dis_ref[...]
         + b_ref[...])
    oh = oh_ref[...]
    dn = (((0,), (0,)), ((), ()))
    s_ref[...] += lax.dot_general(oh, a, dn,
                                  preferred_element_type=jnp.float32)
    q_ref[...] += lax.dot_general(oh, a * a, dn,
                                  preferred_element_type=jnp.float32)


def _agg_stats(eacc, dis2d, onehot, b):
    return pl.pallas_call(
        _stats_body,
        grid=(NB,),
        in_specs=[pl.BlockSpec((2, BLK, 128), lambda i: (0, i, 0)),
                  pl.BlockSpec((BLK, 1), lambda i: (i, 0)),
                  pl.BlockSpec((BLK, G), lambda i: (i, 0)),
                  pl.BlockSpec((1, H), lambda i: (0, 0))],
        out_specs=[pl.BlockSpec((G, H), lambda i: (0, 0)),
                   pl.BlockSpec((G, H), lambda i: (0, 0))],
        out_shape=[jax.ShapeDtypeStruct((G, H), jnp.float32),
                   jax.ShapeDtypeStruct((G, H), jnp.float32)],
    )(eacc, dis2d, onehot, b)


def _hnorm_body(e_ref, dis_ref, oh_ref, b_ref, a_ref, bb_ref,
                h_ref, s_ref, q_ref):
    @pl.when(pl.program_id(0) == 0)
    def _():
        s_ref[...] = jnp.zeros_like(s_ref)
        q_ref[...] = jnp.zeros_like(q_ref)
    oh = oh_ref[...]
    a = (jnp.concatenate([e_ref[0], e_ref[1]], axis=1) * dis_ref[...]
         + b_ref[...])
    ae = jnp.dot(oh, a_ref[...], preferred_element_type=jnp.float32)
    be = jnp.dot(oh, bb_ref[...], preferred_element_type=jnp.float32)
    hh = _lrelu(ae * a + be)
    h_ref[...] = hh
    dn = (((0,), (0,)), ((), ()))
    s_ref[...] += lax.dot_general(oh, hh, dn,
                                  preferred_element_type=jnp.float32)
    q_ref[...] += lax.dot_general(oh, hh * hh, dn,
                                  preferred_element_type=jnp.float32)


def _hnorm(eacc, dis2d, onehot, b, A, B):
    return pl.pallas_call(
        _hnorm_body,
        grid=(NB,),
        in_specs=[pl.BlockSpec((2, BLK, 128), lambda i: (0, i, 0)),
                  pl.BlockSpec((BLK, 1), lambda i: (i, 0)),
                  pl.BlockSpec((BLK, G), lambda i: (i, 0)),
                  pl.BlockSpec((1, H), lambda i: (0, 0)),
                  pl.BlockSpec((G, H), lambda i: (0, 0)),
                  pl.BlockSpec((G, H), lambda i: (0, 0))],
        out_specs=[pl.BlockSpec((BLK, H), lambda i: (i, 0)),
                   pl.BlockSpec((G, H), lambda i: (0, 0)),
                   pl.BlockSpec((G, H), lambda i: (0, 0))],
        out_shape=[jax.ShapeDtypeStruct((NPD, H), jnp.float32),
                   jax.ShapeDtypeStruct((G, H), jnp.float32),
                   jax.ShapeDtypeStruct((G, H), jnp.float32)],
    )(eacc, dis2d, onehot, b, A, B)


def _mf_body(h_ref, oh_ref, a_ref, bb_ref, s_ref):
    @pl.when(pl.program_id(0) == 0)
    def _():
        s_ref[...] = jnp.zeros_like(s_ref)
    oh = oh_ref[...]
    ae = jnp.dot(oh, a_ref[...], preferred_element_type=jnp.float32)
    be = jnp.dot(oh, bb_ref[...], preferred_element_type=jnp.float32)
    mx = _lrelu(ae * h_ref[...] + be)
    dn = (((0,), (0,)), ((), ()))
    s_ref[...] += lax.dot_general(oh, mx, dn,
                                  preferred_element_type=jnp.float32)


def _mf_sum(h, onehot, A, B):
    return pl.pallas_call(
        _mf_body,
        grid=(NB,),
        in_specs=[pl.BlockSpec((BLK, H), lambda i: (i, 0)),
                  pl.BlockSpec((BLK, G), lambda i: (i, 0)),
                  pl.BlockSpec((G, H), lambda i: (0, 0)),
                  pl.BlockSpec((G, H), lambda i: (0, 0))],
        out_specs=pl.BlockSpec((G, H), lambda i: (0, 0)),
        out_shape=jax.ShapeDtypeStruct((G, H), jnp.float32),
    )(h, onehot, A, B)


def _merge_body(gfmf_ref, w_ref, b_ref, o_ref):
    acc = jnp.zeros((G, H), jnp.float32)
    for i in range(L):
        m = _lrelu(jnp.dot(gfmf_ref[i], w_ref[...],
                           preferred_element_type=jnp.float32) + b_ref[...])
        acc = acc + m
    o_ref[...] = acc / L


def _merge(gfmf, w, b):
    return pl.pallas_call(
        _merge_body,
        in_specs=[pl.BlockSpec((L, G, 2 * H), lambda: (0, 0, 0)),
                  pl.BlockSpec((2 * H, H), lambda: (0, 0)),
                  pl.BlockSpec((1, H), lambda: (0, 0))],
        out_specs=pl.BlockSpec((G, H), lambda: (0, 0)),
        out_shape=jax.ShapeDtypeStruct((G, H), jnp.float32),
    )(gfmf, w, b)


def _gn_coeffs(s, q, counts, w, b, ms):
    mean = s / counts[:, None]
    var = q / counts[:, None] - (2.0 * ms - ms * ms) * mean * mean
    a = w / jnp.sqrt(var + EPS)
    bb = b - a * ms * mean
    return a, bb


def kernel(x, params, edge_index, batch):
    src = edge_index[0].astype(jnp.int32)
    dst = edge_index[1].astype(jnp.int32)
    batch = batch.astype(jnp.int32)

    batchp = jnp.concatenate([batch, jnp.full((NPD - N,), G, jnp.int32)])
    onehot = (batchp[:, None] == jnp.arange(G, dtype=jnp.int32)[None, :])
    onehot = onehot.astype(jnp.float32)
    counts = jnp.maximum(jnp.sum(onehot, axis=0), 1.0)

    deg = jnp.zeros((N,), jnp.float32).at[dst].add(1.0) + 1.0
    dis = jnp.concatenate([deg ** -0.5, jnp.zeros((NPD - N,), jnp.float32)])
    dis2d = dis[:, None]

    pad = EP - (E + N)
    sl = jnp.arange(N, dtype=jnp.int32)
    srcp = jnp.concatenate([src, sl, jnp.zeros((pad,), jnp.int32)])
    dstp = jnp.concatenate(
        [dst, sl, N + (jnp.arange(pad, dtype=jnp.int32) % (NPD - N))])
    srcr = srcp.reshape(16, CH, 128)
    dstr = dstp.reshape(16, CH, 128)
    dummy_src = jnp.zeros((16, 2, 128), jnp.int32)
    dummy_dst = jnp.full((16, 2, 128), N, jnp.int32)
    srcr = jnp.concatenate([srcr, dummy_src], axis=1)
    dstr = jnp.concatenate([dstr, dummy_dst], axis=1)
    srci = jnp.stack([srcr, srcr + NPD], axis=0)

    h = jnp.concatenate([x, jnp.zeros((NPD - N, D), x.dtype)], axis=0)
    gfmf = []
    for i in range(L):
        zt = _matmul_scaled(h, dis2d, params[f'conv{i}_W'])
        eacc = _edge_agg(zt.reshape(2 * NPD, 128), srci, dstr)

        b = params[f'conv{i}_b'].reshape(1, H)
        s1, q1 = _agg_stats(eacc, dis2d, onehot, b)
        A1, B1 = _gn_coeffs(s1, q1, counts, params[f'norm{i}_w'],
                            params[f'norm{i}_b'], params[f'norm{i}_ms'])
        h, s2, q2 = _hnorm(eacc, dis2d, onehot, b, A1, B1)
        A2, B2 = _gn_coeffs(s2, q2, counts, params[f'mstnorm{i}_w'],
                            params[f'mstnorm{i}_b'], params[f'mstnorm{i}_ms'])
        smf = _mf_sum(h, onehot, A2, B2)
        gf = s2 / counts[:, None]
        mf = smf / counts[:, None]
        gfmf.append(jnp.concatenate([gf, mf], axis=1))

    out = _merge(jnp.stack(gfmf, 0), params['merge_W'],
                 params['merge_b'].reshape(1, H))
    return out


# speedup vs baseline: 1.5754x; 1.5754x over previous
import functools

import jax
import jax.numpy as jnp
from jax import lax
from jax.experimental import pallas as pl
from jax.experimental.pallas import tpu as pltpu
from jax.experimental.pallas import tpu_sc as plsc

N = 10000
E = 160000
D = 256
H = 256
G = 64
L = 3
EPS = 1e-5
NEG = 0.01

NPD = 10240
BLK = 1280
NB = NPD // BLK

CH = 79
EP = 16 * CH * 128
STRIPE = NPD // 16


def _lrelu(t):
    return jnp.where(t >= 0, t, NEG * t)


def _edge_body(ztf, srci, dsti, out, src_v, dra, drb, bufa, bufb, acc,
               sema, semb, semda, semdb):
    c = lax.axis_index("c")
    s = lax.axis_index("s")
    pltpu.sync_copy(srci.at[c, s], src_v)

    base = s * STRIPE
    for k in range(5):
        sl = pl.ds(base + 128 * k, 128)
        pltpu.sync_copy(ztf.at[pl.ds(c * NPD + base + 128 * k, 128)], bufa)
        pltpu.sync_copy(bufa, acc.at[sl])
    plsc.subcore_barrier()

    pltpu.async_copy(ztf.at[src_v.at[0]], bufa, sema)
    pltpu.async_copy(dsti.at[s, pl.ds(0, 1)], dra, semda)

    def body(j, carry):
        pltpu.async_copy(ztf.at[src_v.at[2 * j + 1]], bufb, semb)
        pltpu.async_copy(dsti.at[s, pl.ds(2 * j + 1, 1)], drb, semdb)
        pltpu.make_async_copy(ztf.at[src_v.at[2 * j]], bufa, sema).wait()
        pltpu.make_async_copy(dsti.at[s, pl.ds(2 * j, 1)], dra, semda).wait()
        pltpu.sync_copy(bufa, acc.at[dra.at[0]], add=True)
        pltpu.async_copy(ztf.at[src_v.at[2 * j + 2]], bufa, sema)
        pltpu.async_copy(dsti.at[s, pl.ds(2 * j + 2, 1)], dra, semda)
        pltpu.make_async_copy(ztf.at[src_v.at[2 * j + 1]], bufb, semb).wait()
        pltpu.make_async_copy(dsti.at[s, pl.ds(2 * j + 1, 1)], drb,
                              semdb).wait()
        pltpu.sync_copy(bufb, acc.at[drb.at[0]], add=True)
        return carry
    lax.fori_loop(0, CH // 2, body, 0)
    pltpu.make_async_copy(ztf.at[src_v.at[CH]], bufa, sema).wait()
    pltpu.make_async_copy(dsti.at[s, pl.ds(CH, 1)], dra, semda).wait()
    plsc.subcore_barrier()

    for k in range(5):
        sl = pl.ds(base + 128 * k, 128)
        pltpu.sync_copy(acc.at[sl], bufa)
        pltpu.sync_copy(bufa, out.at[c, sl])


def _edge_agg(ztf, srci, dsti):
    mesh = plsc.VectorSubcoreMesh(core_axis_name="c", subcore_axis_name="s",
                                  num_cores=2, num_subcores=16)
    return pl.kernel(
        _edge_body,
        out_type=jax.ShapeDtypeStruct((2, NPD, 128), jnp.float32),
        mesh=mesh,
        scratch_types=[
            pltpu.VMEM((CH + 1, 128), jnp.int32),
            pltpu.VMEM((1, 128), jnp.int32),
            pltpu.VMEM((1, 128), jnp.int32),
            pltpu.VMEM((128, 128), jnp.float32),
            pltpu.VMEM((128, 128), jnp.float32),
            pltpu.VMEM_SHARED((NPD, 128), jnp.float32),
            pltpu.SemaphoreType.DMA,
            pltpu.SemaphoreType.DMA,
            pltpu.SemaphoreType.DMA,
            pltpu.SemaphoreType.DMA,
        ],
    )(ztf, srci, dsti)


def _mmz_body(h_ref, dis_ref, w_ref, o_ref):
    ht = h_ref[...] * dis_ref[...]
    z = jnp.dot(ht, w_ref[...], preferred_element_type=jnp.float32)
    o_ref[0] = z[:, :128]
    o_ref[1] = z[:, 128:]


def _matmul_scaled(h, dis2d, w):
    return pl.pallas_call(
        _mmz_body,
        grid=(NB,),
        in_specs=[pl.BlockSpec((BLK, D), lambda i: (i, 0)),
                  pl.BlockSpec((BLK, 1), lambda i: (i, 0)),
                  pl.BlockSpec((D, H), lambda i: (0, 0))],
        out_specs=pl.BlockSpec((2, BLK, 128), lambda i: (0, i, 0)),
        out_shape=jax.ShapeDtypeStruct((2, NPD, 128), jnp.float32),
    )(h, dis2d, w)


def _stats_body(e_ref, dis_ref, oh_ref, b_ref, s_ref, q_ref):
    @pl.when(pl.program_id(0) == 0)
    def _():
        s_ref[...] = jnp.zeros_like(s_ref)
        q_ref[...] = jnp.zeros_like(q_ref)
    a = (jnp.concatenate([e_ref[0], e_ref[1]], axis=1) * dis_ref[...]
         + b_ref[...])
    oh = oh_ref[...]
    dn = (((0,), (0,)), ((), ()))
    s_ref[...] += lax.dot_general(oh, a, dn,
                                  preferred_element_type=jnp.float32)
    q_ref[...] += lax.dot_general(oh, a * a, dn,
                                  preferred_element_type=jnp.float32)


def _agg_stats(eacc, dis2d, onehot, b):
    return pl.pallas_call(
        _stats_body,
        grid=(NB,),
        in_specs=[pl.BlockSpec((2, BLK, 128), lambda i: (0, i, 0)),
                  pl.BlockSpec((BLK, 1), lambda i: (i, 0)),
                  pl.BlockSpec((BLK, G), lambda i: (i, 0)),
                  pl.BlockSpec((1, H), lambda i: (0, 0))],
        out_specs=[pl.BlockSpec((G, H), lambda i: (0, 0)),
                   pl.BlockSpec((G, H), lambda i: (0, 0))],
        out_shape=[jax.ShapeDtypeStruct((G, H), jnp.float32),
                   jax.ShapeDtypeStruct((G, H), jnp.float32)],
    )(eacc, dis2d, onehot, b)


def _hnorm_body(e_ref, dis_ref, oh_ref, b_ref, a_ref, bb_ref,
                h_ref, s_ref, q_ref):
    @pl.when(pl.program_id(0) == 0)
    def _():
        s_ref[...] = jnp.zeros_like(s_ref)
        q_ref[...] = jnp.zeros_like(q_ref)
    oh = oh_ref[...]
    a = (jnp.concatenate([e_ref[0], e_ref[1]], axis=1) * dis_ref[...]
         + b_ref[...])
    ae = jnp.dot(oh, a_ref[...], preferred_element_type=jnp.float32)
    be = jnp.dot(oh, bb_ref[...], preferred_element_type=jnp.float32)
    hh = _lrelu(ae * a + be)
    h_ref[...] = hh
    dn = (((0,), (0,)), ((), ()))
    s_ref[...] += lax.dot_general(oh, hh, dn,
                                  preferred_element_type=jnp.float32)
    q_ref[...] += lax.dot_general(oh, hh * hh, dn,
                                  preferred_element_type=jnp.float32)


def _hnorm(eacc, dis2d, onehot, b, A, B):
    return pl.pallas_call(
        _hnorm_body,
        grid=(NB,),
        in_specs=[pl.BlockSpec((2, BLK, 128), lambda i: (0, i, 0)),
                  pl.BlockSpec((BLK, 1), lambda i: (i, 0)),
                  pl.BlockSpec((BLK, G), lambda i: (i, 0)),
                  pl.BlockSpec((1, H), lambda i: (0, 0)),
                  pl.BlockSpec((G, H), lambda i: (0, 0)),
                  pl.BlockSpec((G, H), lambda i: (0, 0))],
        out_specs=[pl.BlockSpec((BLK, H), lambda i: (i, 0)),
                   pl.BlockSpec((G, H), lambda i: (0, 0)),
                   pl.BlockSpec((G, H), lambda i: (0, 0))],
        out_shape=[jax.ShapeDtypeStruct((NPD, H), jnp.float32),
                   jax.ShapeDtypeStruct((G, H), jnp.float32),
                   jax.ShapeDtypeStruct((G, H), jnp.float32)],
    )(eacc, dis2d, onehot, b, A, B)


def _mf_body(h_ref, oh_ref, a_ref, bb_ref, s_ref):
    @pl.when(pl.program_id(0) == 0)
    def _():
        s_ref[...] = jnp.zeros_like(s_ref)
    oh = oh_ref[...]
    ae = jnp.dot(oh, a_ref[...], preferred_element_type=jnp.float32)
    be = jnp.dot(oh, bb_ref[...], preferred_element_type=jnp.float32)
    mx = _lrelu(ae * h_ref[...] + be)
    dn = (((0,), (0,)), ((), ()))
    s_ref[...] += lax.dot_general(oh, mx, dn,
                                  preferred_element_type=jnp.float32)


def _mf_sum(h, onehot, A, B):
    return pl.pallas_call(
        _mf_body,
        grid=(NB,),
        in_specs=[pl.BlockSpec((BLK, H), lambda i: (i, 0)),
                  pl.BlockSpec((BLK, G), lambda i: (i, 0)),
                  pl.BlockSpec((G, H), lambda i: (0, 0)),
                  pl.BlockSpec((G, H), lambda i: (0, 0))],
        out_specs=pl.BlockSpec((G, H), lambda i: (0, 0)),
        out_shape=jax.ShapeDtypeStruct((G, H), jnp.float32),
    )(h, onehot, A, B)


def _merge_body(gfmf_ref, w_ref, b_ref, o_ref):
    acc = jnp.zeros((G, H), jnp.float32)
    for i in range(L):
        m = _lrelu(jnp.dot(gfmf_ref[i], w_ref[...],
                           preferred_element_type=jnp.float32) + b_ref[...])
        acc = acc + m
    o_ref[...] = acc / L


def _merge(gfmf, w, b):
    return pl.pallas_call(
        _merge_body,
        in_specs=[pl.BlockSpec((L, G, 2 * H), lambda: (0, 0, 0)),
                  pl.BlockSpec((2 * H, H), lambda: (0, 0)),
                  pl.BlockSpec((1, H), lambda: (0, 0))],
        out_specs=pl.BlockSpec((G, H), lambda: (0, 0)),
        out_shape=jax.ShapeDtypeStruct((G, H), jnp.float32),
    )(gfmf, w, b)


def _gn_coeffs(s, q, counts, w, b, ms):
    mean = s / counts[:, None]
    var = q / counts[:, None] - (2.0 * ms - ms * ms) * mean * mean
    a = w / jnp.sqrt(var + EPS)
    bb = b - a * ms * mean
    return a, bb


def kernel(x, params, edge_index, batch):
    src = edge_index[0].astype(jnp.int32)
    dst = edge_index[1].astype(jnp.int32)
    batch = batch.astype(jnp.int32)

    batchp = jnp.concatenate([batch, jnp.full((NPD - N,), G, jnp.int32)])
    onehot = (batchp[:, None] == jnp.arange(G, dtype=jnp.int32)[None, :])
    onehot = onehot.astype(jnp.float32)
    counts = jnp.maximum(jnp.sum(onehot, axis=0), 1.0)

    deg = jnp.zeros((N,), jnp.float32).at[dst].add(1.0) + 1.0
    dis = jnp.concatenate([deg ** -0.5, jnp.zeros((NPD - N,), jnp.float32)])
    dis2d = dis[:, None]

    pad = EP - E
    srcp = jnp.concatenate([src, jnp.zeros((pad,), jnp.int32)])
    dstp = jnp.concatenate(
        [dst, N + (jnp.arange(pad, dtype=jnp.int32) % (NPD - N))])
    srcr = srcp.reshape(16, CH, 128)
    dstr = dstp.reshape(16, CH, 128)
    dummy_src = jnp.zeros((16, 1, 128), jnp.int32)
    dummy_dst = jnp.full((16, 1, 128), N, jnp.int32)
    srcr = jnp.concatenate([srcr, dummy_src], axis=1)
    dstr = jnp.concatenate([dstr, dummy_dst], axis=1)
    srci = jnp.stack([srcr, srcr + NPD], axis=0)

    h = jnp.concatenate([x, jnp.zeros((NPD - N, D), x.dtype)], axis=0)
    gfmf = []
    for i in range(L):
        zt = _matmul_scaled(h, dis2d, params[f'conv{i}_W'])
        eacc = _edge_agg(zt.reshape(2 * NPD, 128), srci, dstr)

        b = params[f'conv{i}_b'].reshape(1, H)
        s1, q1 = _agg_stats(eacc, dis2d, onehot, b)
        A1, B1 = _gn_coeffs(s1, q1, counts, params[f'norm{i}_w'],
                            params[f'norm{i}_b'], params[f'norm{i}_ms'])
        h, s2, q2 = _hnorm(eacc, dis2d, onehot, b, A1, B1)
        A2, B2 = _gn_coeffs(s2, q2, counts, params[f'mstnorm{i}_w'],
                            params[f'mstnorm{i}_b'], params[f'mstnorm{i}_ms'])
        smf = _mf_sum(h, onehot, A2, B2)
        gf = s2 / counts[:, None]
        mf = smf / counts[:, None]
        gfmf.append(jnp.concatenate([gf, mf], axis=1))

    out = _merge(jnp.stack(gfmf, 0), params['merge_W'],
                 params['merge_b'].reshape(1, H))
    return out


# speedup vs baseline: 1.6466x; 1.0452x over previous
import functools

import jax
import jax.numpy as jnp
from jax import lax
from jax.experimental import pallas as pl
from jax.experimental.pallas import tpu as pltpu
from jax.experimental.pallas import tpu_sc as plsc

N = 10000
E = 160000
D = 256
H = 256
G = 64
L = 3
EPS = 1e-5
NEG = 0.01

NPD = 10240
BLK = 1280
NB = NPD // BLK

CH = 79
EP = 16 * CH * 128
STRIPE = NPD // 16


def _lrelu(t):
    return jnp.where(t >= 0, t, NEG * t)


def _edge_body(ztf, srci, dsti, out, src_v, dra, drb, bufa, bufb, acc,
               sema, semb, semda, semdb):
    c = lax.axis_index("c")
    s = lax.axis_index("s")
    pltpu.sync_copy(srci.at[c, s], src_v)

    base = s * STRIPE
    for k in range(5):
        sl = pl.ds(base + 128 * k, 128)
        pltpu.sync_copy(ztf.at[pl.ds(c * NPD + base + 128 * k, 128)], bufa)
        pltpu.sync_copy(bufa, acc.at[sl])
    plsc.subcore_barrier()

    pltpu.async_copy(ztf.at[src_v.at[0]], bufa, sema)
    pltpu.async_copy(dsti.at[s, pl.ds(0, 1)], dra, semda)

    def body(j, carry):
        pltpu.async_copy(ztf.at[src_v.at[2 * j + 1]], bufb, semb)
        pltpu.async_copy(dsti.at[s, pl.ds(2 * j + 1, 1)], drb, semdb)
        pltpu.make_async_copy(ztf.at[src_v.at[2 * j]], bufa, sema).wait()
        pltpu.make_async_copy(dsti.at[s, pl.ds(2 * j, 1)], dra, semda).wait()
        pltpu.sync_copy(bufa, acc.at[dra.at[0]], add=True)
        pltpu.async_copy(ztf.at[src_v.at[2 * j + 2]], bufa, sema)
        pltpu.async_copy(dsti.at[s, pl.ds(2 * j + 2, 1)], dra, semda)
        pltpu.make_async_copy(ztf.at[src_v.at[2 * j + 1]], bufb, semb).wait()
        pltpu.make_async_copy(dsti.at[s, pl.ds(2 * j + 1, 1)], drb,
                              semdb).wait()
        pltpu.sync_copy(bufb, acc.at[drb.at[0]], add=True)
        return carry
    lax.fori_loop(0, CH // 2, body, 0)
    pltpu.make_async_copy(ztf.at[src_v.at[CH]], bufa, sema).wait()
    pltpu.make_async_copy(dsti.at[s, pl.ds(CH, 1)], dra, semda).wait()
    plsc.subcore_barrier()

    for k in range(5):
        sl = pl.ds(base + 128 * k, 128)
        pltpu.sync_copy(acc.at[sl], bufa)
        pltpu.sync_copy(bufa, out.at[c, sl])


def _edge_agg(ztf, srci, dsti):
    mesh = plsc.VectorSubcoreMesh(core_axis_name="c", subcore_axis_name="s",
                                  num_cores=2, num_subcores=16)
    return pl.kernel(
        _edge_body,
        out_type=jax.ShapeDtypeStruct((2, NPD, 128), jnp.float32),
        mesh=mesh,
        scratch_types=[
            pltpu.VMEM((CH + 1, 128), jnp.int32),
            pltpu.VMEM((1, 128), jnp.int32),
            pltpu.VMEM((1, 128), jnp.int32),
            pltpu.VMEM((128, 128), jnp.float32),
            pltpu.VMEM((128, 128), jnp.float32),
            pltpu.VMEM_SHARED((NPD, 128), jnp.float32),
            pltpu.SemaphoreType.DMA,
            pltpu.SemaphoreType.DMA,
            pltpu.SemaphoreType.DMA,
            pltpu.SemaphoreType.DMA,
        ],
    )(ztf, srci, dsti)


def _mmz_body(h_ref, dis_ref, w_ref, o_ref):
    ht = h_ref[...] * dis_ref[...]
    z = jnp.dot(ht, w_ref[...], preferred_element_type=jnp.float32)
    o_ref[0] = z[:, :128]
    o_ref[1] = z[:, 128:]


def _matmul_scaled(h, dis2d, w):
    return pl.pallas_call(
        _mmz_body,
        grid=(NB,),
        in_specs=[pl.BlockSpec((BLK, D), lambda i: (i, 0)),
                  pl.BlockSpec((BLK, 1), lambda i: (i, 0)),
                  pl.BlockSpec((D, H), lambda i: (0, 0))],
        out_specs=pl.BlockSpec((2, BLK, 128), lambda i: (0, i, 0)),
        out_shape=jax.ShapeDtypeStruct((2, NPD, 128), jnp.float32),
    )(h, dis2d, w)


def _mega_body(e_ref, dis_ref, bt_ref, b_ref, w1_ref, b1_ref, ms1_ref,
               w2_ref, b2_ref, ms2_ref, wn_ref,
               zt_ref, s2_ref, smf_ref, cnt_ref,
               ac_ref, hc_ref, s1_ref, q1_ref, q2_ref,
               a1_ref, bb1_ref, a2_ref, bb2_ref):
    p = pl.program_id(0)
    i = pl.program_id(1)
    dn = (((0,), (0,)), ((), ()))

    @pl.when((p == 0) & (i == 0))
    def _():
        s1_ref[...] = jnp.zeros_like(s1_ref)
        q1_ref[...] = jnp.zeros_like(q1_ref)
        q2_ref[...] = jnp.zeros_like(q2_ref)
        s2_ref[...] = jnp.zeros_like(s2_ref)
        smf_ref[...] = jnp.zeros_like(smf_ref)
        cnt_ref[...] = jnp.zeros_like(cnt_ref)

    oh = (bt_ref[...] == lax.broadcasted_iota(jnp.int32, (BLK, G), 1))
    oh = oh.astype(jnp.float32)
    row = pl.ds(0, BLK)

    @pl.when(p == 0)
    def _():
        a = (jnp.concatenate([e_ref[0], e_ref[1]], axis=1) * dis_ref[...]
             + b_ref[...])
        ac_ref[pl.ds(i * BLK, BLK), :] = a
        s1_ref[...] += lax.dot_general(oh, a, dn,
                                       preferred_element_type=jnp.float32)
        q1_ref[...] += lax.dot_general(oh, a * a, dn,
                                       preferred_element_type=jnp.float32)
        cnt_ref[...] += jnp.sum(oh, axis=0)[:, None]

    @pl.when((p == 1) & (i == 0))
    def _():
        cnt = jnp.maximum(cnt_ref[...], 1.0)
        mean = s1_ref[...] / cnt
        ms = ms1_ref[...]
        var = q1_ref[...] / cnt - (2.0 * ms - ms * ms) * mean * mean
        aa = w1_ref[...] / jnp.sqrt(var + EPS)
        a1_ref[...] = aa
        bb1_ref[...] = b1_ref[...] - aa * ms * mean

    @pl.when(p == 1)
    def _():
        a = ac_ref[pl.ds(i * BLK, BLK), :]
        ae = jnp.dot(oh, a1_ref[...], preferred_element_type=jnp.float32)
        be = jnp.dot(oh, bb1_ref[...], preferred_element_type=jnp.float32)
        hh = _lrelu(ae * a + be)
        hc_ref[pl.ds(i * BLK, BLK), :] = hh
        s2_ref[...] += lax.dot_general(oh, hh, dn,
                                       preferred_element_type=jnp.float32)
        q2_ref[...] += lax.dot_general(oh, hh * hh, dn,
                                       preferred_element_type=jnp.float32)

    @pl.when((p == 2) & (i == 0))
    def _():
        cnt = jnp.maximum(cnt_ref[...], 1.0)
        mean = s2_ref[...] / cnt
        ms = ms2_ref[...]
        var = q2_ref[...] / cnt - (2.0 * ms - ms * ms) * mean * mean
        aa = w2_ref[...] / jnp.sqrt(var + EPS)
        a2_ref[...] = aa
        bb2_ref[...] = b2_ref[...] - aa * ms * mean

    @pl.when(p == 2)
    def _():
        hh = hc_ref[pl.ds(i * BLK, BLK), :]
        ae = jnp.dot(oh, a2_ref[...], preferred_element_type=jnp.float32)
        be = jnp.dot(oh, bb2_ref[...], preferred_element_type=jnp.float32)
        mx = _lrelu(ae * hh + be)
        smf_ref[...] += lax.dot_general(oh, mx, dn,
                                        preferred_element_type=jnp.float32)
        zt = jnp.dot(hh * dis_ref[...], wn_ref[...],
                     preferred_element_type=jnp.float32)
        zt_ref[0] = zt[:, :128]
        zt_ref[1] = zt[:, 128:]


def _mega(eacc, dis2d, batch2d, b, w1, b1, ms1, w2, b2, ms2, wn):
    vec = pl.BlockSpec((1, H), lambda p, i: (0, 0))
    return pl.pallas_call(
        _mega_body,
        grid=(3, NB),
        in_specs=[
            pl.BlockSpec((2, BLK, 128),
                         lambda p, i: (0, jnp.where(p == 0, i, 0), 0)),
            pl.BlockSpec((BLK, 1),
                         lambda p, i: (jnp.where(p == 1, 0, i), 0)),
            pl.BlockSpec((BLK, 1), lambda p, i: (i, 0)),
            vec, vec, vec, vec, vec, vec, vec,
            pl.BlockSpec((H, H), lambda p, i: (0, 0)),
        ],
        out_specs=[
            pl.BlockSpec((2, BLK, 128),
                         lambda p, i: (0, jnp.where(p == 2, i, 0), 0)),
            pl.BlockSpec((G, H), lambda p, i: (0, 0)),
            pl.BlockSpec((G, H), lambda p, i: (0, 0)),
            pl.BlockSpec((G, 1), lambda p, i: (0, 0)),
        ],
        out_shape=[jax.ShapeDtypeStruct((2, NPD, 128), jnp.float32),
                   jax.ShapeDtypeStruct((G, H), jnp.float32),
                   jax.ShapeDtypeStruct((G, H), jnp.float32),
                   jax.ShapeDtypeStruct((G, 1), jnp.float32)],
        scratch_shapes=[pltpu.VMEM((NPD, H), jnp.float32),
                        pltpu.VMEM((NPD, H), jnp.float32),
                        pltpu.VMEM((G, H), jnp.float32),
                        pltpu.VMEM((G, H), jnp.float32),
                        pltpu.VMEM((G, H), jnp.float32),
                        pltpu.VMEM((G, H), jnp.float32),
                        pltpu.VMEM((G, H), jnp.float32),
                        pltpu.VMEM((G, H), jnp.float32),
                        pltpu.VMEM((G, H), jnp.float32)],
    )(eacc, dis2d, batch2d, b, w1, b1, ms1, w2, b2, ms2, wn)


def _merge_body(gfmf_ref, w_ref, b_ref, o_ref):
    acc = jnp.zeros((G, H), jnp.float32)
    for i in range(L):
        m = _lrelu(jnp.dot(gfmf_ref[i], w_ref[...],
                           preferred_element_type=jnp.float32) + b_ref[...])
        acc = acc + m
    o_ref[...] = acc / L


def _merge(gfmf, w, b):
    return pl.pallas_call(
        _merge_body,
        in_specs=[pl.BlockSpec((L, G, 2 * H), lambda: (0, 0, 0)),
                  pl.BlockSpec((2 * H, H), lambda: (0, 0)),
                  pl.BlockSpec((1, H), lambda: (0, 0))],
        out_specs=pl.BlockSpec((G, H), lambda: (0, 0)),
        out_shape=jax.ShapeDtypeStruct((G, H), jnp.float32),
    )(gfmf, w, b)


def _gn_coeffs(s, q, counts, w, b, ms):
    mean = s / counts[:, None]
    var = q / counts[:, None] - (2.0 * ms - ms * ms) * mean * mean
    a = w / jnp.sqrt(var + EPS)
    bb = b - a * ms * mean
    return a, bb


def kernel(x, params, edge_index, batch):
    src = edge_index[0].astype(jnp.int32)
    dst = edge_index[1].astype(jnp.int32)
    batch = batch.astype(jnp.int32)

    batchp = jnp.concatenate([batch, jnp.full((NPD - N,), G, jnp.int32)])
    batch2d = batchp[:, None]

    deg = jnp.zeros((N,), jnp.float32).at[dst].add(1.0) + 1.0
    dis = jnp.concatenate([deg ** -0.5, jnp.zeros((NPD - N,), jnp.float32)])
    dis2d = dis[:, None]

    pad = EP - E
    srcp = jnp.concatenate([src, jnp.zeros((pad,), jnp.int32)])
    dstp = jnp.concatenate(
        [dst, N + (jnp.arange(pad, dtype=jnp.int32) % (NPD - N))])
    srcr = srcp.reshape(16, CH, 128)
    dstr = dstp.reshape(16, CH, 128)
    dummy_src = jnp.zeros((16, 1, 128), jnp.int32)
    dummy_dst = jnp.full((16, 1, 128), N, jnp.int32)
    srcr = jnp.concatenate([srcr, dummy_src], axis=1)
    dstr = jnp.concatenate([dstr, dummy_dst], axis=1)
    srci = jnp.stack([srcr, srcr + NPD], axis=0)

    xp = jnp.concatenate([x, jnp.zeros((NPD - N, D), x.dtype)], axis=0)
    zt = _matmul_scaled(xp, dis2d, params['conv0_W'])
    gfmf = []
    counts = None
    for i in range(L):
        eacc = _edge_agg(zt.reshape(2 * NPD, 128), srci, dstr)
        wn = params[f'conv{i + 1}_W'] if i + 1 < L else params['conv0_W'][:H]
        zt, s2, smf, cnt = _mega(
            eacc, dis2d, batch2d, params[f'conv{i}_b'].reshape(1, H),
            params[f'norm{i}_w'].reshape(1, H),
            params[f'norm{i}_b'].reshape(1, H),
            params[f'norm{i}_ms'].reshape(1, H),
            params[f'mstnorm{i}_w'].reshape(1, H),
            params[f'mstnorm{i}_b'].reshape(1, H),
            params[f'mstnorm{i}_ms'].reshape(1, H), wn)
        counts = jnp.maximum(cnt, 1.0)
        gfmf.append(jnp.concatenate([s2 / counts, smf / counts], axis=1))

    out = _merge(jnp.stack(gfmf, 0), params['merge_W'],
                 params['merge_b'].reshape(1, H))
    return out


# speedup vs baseline: 2.1126x; 1.2830x over previous
import functools

import jax
import jax.numpy as jnp
from jax import lax
from jax.experimental import pallas as pl
from jax.experimental.pallas import tpu as pltpu
from jax.experimental.pallas import tpu_sc as plsc

N = 10000
E = 160000
D = 256
H = 256
G = 64
L = 3
EPS = 1e-5
NEG = 0.01

NPD = 10240
BLK = 1280
NB = NPD // BLK

CH = 79
EP = 16 * CH * 128
STRIPE = NPD // 16


def _lrelu(t):
    return jnp.where(t >= 0, t, NEG * t)


def _edge_body(ztf, srci, dsti, out, src_v, dra, drb, bufa, bufb, acc,
               sema, semb, semda, semdb):
    c = lax.axis_index("c")
    s = lax.axis_index("s")
    pltpu.sync_copy(srci.at[c, s], src_v)

    base = s * STRIPE
    for k in range(5):
        sl = pl.ds(base + 128 * k, 128)
        pltpu.sync_copy(ztf.at[pl.ds(c * NPD + base + 128 * k, 128)], bufa)
        pltpu.sync_copy(bufa, acc.at[sl])
    plsc.subcore_barrier()

    pltpu.async_copy(ztf.at[src_v.at[0]], bufa, sema)
    pltpu.async_copy(dsti.at[s, pl.ds(0, 1)], dra, semda)

    def body(j, carry):
        pltpu.async_copy(ztf.at[src_v.at[2 * j + 1]], bufb, semb)
        pltpu.async_copy(dsti.at[s, pl.ds(2 * j + 1, 1)], drb, semdb)
        pltpu.make_async_copy(ztf.at[src_v.at[2 * j]], bufa, sema).wait()
        pltpu.make_async_copy(dsti.at[s, pl.ds(2 * j, 1)], dra, semda).wait()
        pltpu.sync_copy(bufa, acc.at[dra.at[0]], add=True)
        pltpu.async_copy(ztf.at[src_v.at[2 * j + 2]], bufa, sema)
        pltpu.async_copy(dsti.at[s, pl.ds(2 * j + 2, 1)], dra, semda)
        pltpu.make_async_copy(ztf.at[src_v.at[2 * j + 1]], bufb, semb).wait()
        pltpu.make_async_copy(dsti.at[s, pl.ds(2 * j + 1, 1)], drb,
                              semdb).wait()
        pltpu.sync_copy(bufb, acc.at[drb.at[0]], add=True)
        return carry
    lax.fori_loop(0, CH // 2, body, 0)
    pltpu.make_async_copy(ztf.at[src_v.at[CH]], bufa, sema).wait()
    pltpu.make_async_copy(dsti.at[s, pl.ds(CH, 1)], dra, semda).wait()
    plsc.subcore_barrier()

    for k in range(5):
        sl = pl.ds(base + 128 * k, 128)
        pltpu.sync_copy(acc.at[sl], bufa)
        pltpu.sync_copy(bufa, out.at[c, sl])


def _edge_agg(ztf, srci, dsti):
    mesh = plsc.VectorSubcoreMesh(core_axis_name="c", subcore_axis_name="s",
                                  num_cores=2, num_subcores=16)
    return pl.kernel(
        _edge_body,
        out_type=jax.ShapeDtypeStruct((2, NPD, 128), jnp.float32),
        mesh=mesh,
        scratch_types=[
            pltpu.VMEM((CH + 1, 128), jnp.int32),
            pltpu.VMEM((1, 128), jnp.int32),
            pltpu.VMEM((1, 128), jnp.int32),
            pltpu.VMEM((128, 128), jnp.float32),
            pltpu.VMEM((128, 128), jnp.float32),
            pltpu.VMEM_SHARED((NPD, 128), jnp.float32),
            pltpu.SemaphoreType.DMA,
            pltpu.SemaphoreType.DMA,
            pltpu.SemaphoreType.DMA,
            pltpu.SemaphoreType.DMA,
        ],
    )(ztf, srci, dsti)


def _deg_body(dsti, out, dra, drb, bufo, accd, semda, semdb):
    c = lax.axis_index("c")
    s = lax.axis_index("s")

    def zrow(r, carry):
        bufo[r, :] = jnp.zeros((16,), jnp.float32)
        return carry
    lax.fori_loop(0, 128, zrow, 0)
    base = s * STRIPE
    for k in range(5):
        pltpu.sync_copy(bufo, accd.at[pl.ds(base + 128 * k, 128)])

    def orow(r, carry):
        bufo[r, :] = jnp.ones((16,), jnp.float32)
        return carry
    lax.fori_loop(0, 128, orow, 0)
    plsc.subcore_barrier()

    pltpu.async_copy(dsti.at[s, pl.ds(0, 1)], dra, semda)

    def body(j, carry):
        pltpu.async_copy(dsti.at[s, pl.ds(2 * j + 1, 1)], drb, semdb)
        pltpu.make_async_copy(dsti.at[s, pl.ds(2 * j, 1)], dra, semda).wait()
        pltpu.sync_copy(bufo, accd.at[dra.at[0]], add=True)
        pltpu.async_copy(dsti.at[s, pl.ds(2 * j + 2, 1)], dra, semda)
        pltpu.make_async_copy(dsti.at[s, pl.ds(2 * j + 1, 1)], drb,
                              semdb).wait()
        pltpu.sync_copy(bufo, accd.at[drb.at[0]], add=True)
        return carry
    lax.fori_loop(0, (CH + 1) // 2, body, 0)
    pltpu.make_async_copy(dsti.at[s, pl.ds(CH + 1, 1)], dra, semda).wait()
    plsc.subcore_barrier()

    for k in range(5):
        sl = pl.ds(base + 128 * k, 128)
        pltpu.sync_copy(accd.at[sl], bufo)
        pltpu.sync_copy(bufo, out.at[c, sl])


def _deg_count(dsti):
    mesh = plsc.VectorSubcoreMesh(core_axis_name="c", subcore_axis_name="s",
                                  num_cores=2, num_subcores=16)
    return pl.kernel(
        _deg_body,
        out_type=jax.ShapeDtypeStruct((2, NPD, 16), jnp.float32),
        mesh=mesh,
        scratch_types=[
            pltpu.VMEM((1, 128), jnp.int32),
            pltpu.VMEM((1, 128), jnp.int32),
            pltpu.VMEM((128, 16), jnp.float32),
            pltpu.VMEM_SHARED((NPD, 16), jnp.float32),
            pltpu.SemaphoreType.DMA,
            pltpu.SemaphoreType.DMA,
        ],
    )(dsti)


def _mmz_body(h_ref, dis_ref, w_ref, o_ref):
    ht = h_ref[...] * dis_ref[...]
    z = jnp.dot(ht, w_ref[...], preferred_element_type=jnp.float32)
    o_ref[0] = z[:, :128]
    o_ref[1] = z[:, 128:]


def _matmul_scaled(h, dis2d, w):
    return pl.pallas_call(
        _mmz_body,
        grid=(NB,),
        in_specs=[pl.BlockSpec((BLK, D), lambda i: (i, 0)),
                  pl.BlockSpec((BLK, 1), lambda i: (i, 0)),
                  pl.BlockSpec((D, H), lambda i: (0, 0))],
        out_specs=pl.BlockSpec((2, BLK, 128), lambda i: (0, i, 0)),
        out_shape=jax.ShapeDtypeStruct((2, NPD, 128), jnp.float32),
    )(h, dis2d, w)


def _mega_body(e_ref, dis_ref, bt_ref, b_ref, w1_ref, b1_ref, ms1_ref,
               w2_ref, b2_ref, ms2_ref, wn_ref,
               zt_ref, s2_ref, smf_ref, cnt_ref,
               ac_ref, hc_ref, s1_ref, q1_ref, q2_ref,
               a1_ref, bb1_ref, a2_ref, bb2_ref):
    p = pl.program_id(0)
    i = pl.program_id(1)
    dn = (((0,), (0,)), ((), ()))

    @pl.when((p == 0) & (i == 0))
    def _():
        s1_ref[...] = jnp.zeros_like(s1_ref)
        q1_ref[...] = jnp.zeros_like(q1_ref)
        q2_ref[...] = jnp.zeros_like(q2_ref)
        s2_ref[...] = jnp.zeros_like(s2_ref)
        smf_ref[...] = jnp.zeros_like(smf_ref)
        cnt_ref[...] = jnp.zeros_like(cnt_ref)

    oh = (bt_ref[...] == lax.broadcasted_iota(jnp.int32, (BLK, G), 1))
    oh = oh.astype(jnp.float32)
    row = pl.ds(0, BLK)

    @pl.when(p == 0)
    def _():
        a = (jnp.concatenate([e_ref[0], e_ref[1]], axis=1) * dis_ref[...]
             + b_ref[...])
        ac_ref[pl.ds(i * BLK, BLK), :] = a
        s1_ref[...] += lax.dot_general(oh, a, dn,
                                       preferred_element_type=jnp.float32)
        q1_ref[...] += lax.dot_general(oh, a * a, dn,
                                       preferred_element_type=jnp.float32)
        cnt_ref[...] += jnp.sum(oh, axis=0)[:, None]

    @pl.when((p == 1) & (i == 0))
    def _():
        cnt = jnp.maximum(cnt_ref[...], 1.0)
        mean = s1_ref[...] / cnt
        ms = ms1_ref[...]
        var = q1_ref[...] / cnt - (2.0 * ms - ms * ms) * mean * mean
        aa = w1_ref[...] / jnp.sqrt(var + EPS)
        a1_ref[...] = aa
        bb1_ref[...] = b1_ref[...] - aa * ms * mean

    @pl.when(p == 1)
    def _():
        a = ac_ref[pl.ds(i * BLK, BLK), :]
        ae = jnp.dot(oh, a1_ref[...], preferred_element_type=jnp.float32)
        be = jnp.dot(oh, bb1_ref[...], preferred_element_type=jnp.float32)
        hh = _lrelu(ae * a + be)
        hc_ref[pl.ds(i * BLK, BLK), :] = hh
        s2_ref[...] += lax.dot_general(oh, hh, dn,
                                       preferred_element_type=jnp.float32)
        q2_ref[...] += lax.dot_general(oh, hh * hh, dn,
                                       preferred_element_type=jnp.float32)

    @pl.when((p == 2) & (i == 0))
    def _():
        cnt = jnp.maximum(cnt_ref[...], 1.0)
        mean = s2_ref[...] / cnt
        ms = ms2_ref[...]
        var = q2_ref[...] / cnt - (2.0 * ms - ms * ms) * mean * mean
        aa = w2_ref[...] / jnp.sqrt(var + EPS)
        a2_ref[...] = aa
        bb2_ref[...] = b2_ref[...] - aa * ms * mean

    @pl.when(p == 2)
    def _():
        hh = hc_ref[pl.ds(i * BLK, BLK), :]
        ae = jnp.dot(oh, a2_ref[...], preferred_element_type=jnp.float32)
        be = jnp.dot(oh, bb2_ref[...], preferred_element_type=jnp.float32)
        mx = _lrelu(ae * hh + be)
        smf_ref[...] += lax.dot_general(oh, mx, dn,
                                        preferred_element_type=jnp.float32)
        zt = jnp.dot(hh * dis_ref[...], wn_ref[...],
                     preferred_element_type=jnp.float32)
        zt_ref[0] = zt[:, :128]
        zt_ref[1] = zt[:, 128:]


def _mega(eacc, dis2d, batch2d, b, w1, b1, ms1, w2, b2, ms2, wn):
    vec = pl.BlockSpec((1, H), lambda p, i: (0, 0))
    return pl.pallas_call(
        _mega_body,
        grid=(3, NB),
        in_specs=[
            pl.BlockSpec((2, BLK, 128),
                         lambda p, i: (0, jnp.where(p == 0, i, 0), 0)),
            pl.BlockSpec((BLK, 1),
                         lambda p, i: (jnp.where(p == 1, 0, i), 0)),
            pl.BlockSpec((BLK, 1), lambda p, i: (i, 0)),
            vec, vec, vec, vec, vec, vec, vec,
            pl.BlockSpec((H, H), lambda p, i: (0, 0)),
        ],
        out_specs=[
            pl.BlockSpec((2, BLK, 128),
                         lambda p, i: (0, jnp.where(p == 2, i, 0), 0)),
            pl.BlockSpec((G, H), lambda p, i: (0, 0)),
            pl.BlockSpec((G, H), lambda p, i: (0, 0)),
            pl.BlockSpec((G, 1), lambda p, i: (0, 0)),
        ],
        out_shape=[jax.ShapeDtypeStruct((2, NPD, 128), jnp.float32),
                   jax.ShapeDtypeStruct((G, H), jnp.float32),
                   jax.ShapeDtypeStruct((G, H), jnp.float32),
                   jax.ShapeDtypeStruct((G, 1), jnp.float32)],
        scratch_shapes=[pltpu.VMEM((NPD, H), jnp.float32),
                        pltpu.VMEM((NPD, H), jnp.float32),
                        pltpu.VMEM((G, H), jnp.float32),
                        pltpu.VMEM((G, H), jnp.float32),
                        pltpu.VMEM((G, H), jnp.float32),
                        pltpu.VMEM((G, H), jnp.float32),
                        pltpu.VMEM((G, H), jnp.float32),
                        pltpu.VMEM((G, H), jnp.float32),
                        pltpu.VMEM((G, H), jnp.float32)],
    )(eacc, dis2d, batch2d, b, w1, b1, ms1, w2, b2, ms2, wn)


def _merge_body(gfmf_ref, w_ref, b_ref, o_ref):
    acc = jnp.zeros((G, H), jnp.float32)
    for i in range(L):
        m = _lrelu(jnp.dot(gfmf_ref[i], w_ref[...],
                           preferred_element_type=jnp.float32) + b_ref[...])
        acc = acc + m
    o_ref[...] = acc / L


def _merge(gfmf, w, b):
    return pl.pallas_call(
        _merge_body,
        in_specs=[pl.BlockSpec((L, G, 2 * H), lambda: (0, 0, 0)),
                  pl.BlockSpec((2 * H, H), lambda: (0, 0)),
                  pl.BlockSpec((1, H), lambda: (0, 0))],
        out_specs=pl.BlockSpec((G, H), lambda: (0, 0)),
        out_shape=jax.ShapeDtypeStruct((G, H), jnp.float32),
    )(gfmf, w, b)


def _gn_coeffs(s, q, counts, w, b, ms):
    mean = s / counts[:, None]
    var = q / counts[:, None] - (2.0 * ms - ms * ms) * mean * mean
    a = w / jnp.sqrt(var + EPS)
    bb = b - a * ms * mean
    return a, bb


def kernel(x, params, edge_index, batch):
    src = edge_index[0].astype(jnp.int32)
    dst = edge_index[1].astype(jnp.int32)
    batch = batch.astype(jnp.int32)

    batchp = jnp.concatenate([batch, jnp.full((NPD - N,), G, jnp.int32)])
    batch2d = batchp[:, None]

    pad = EP - E
    srcp = jnp.concatenate([src, jnp.zeros((pad,), jnp.int32)])
    dstp = jnp.concatenate(
        [dst, N + (jnp.arange(pad, dtype=jnp.int32) % (NPD - N))])
    srcr = srcp.reshape(16, CH, 128)
    dstr = dstp.reshape(16, CH, 128)
    dummy_src = jnp.zeros((16, 2, 128), jnp.int32)
    dummy_dst = jnp.full((16, 2, 128), N, jnp.int32)
    srcr = jnp.concatenate([srcr, dummy_src[:, :1]], axis=1)
    dstr2 = jnp.concatenate([dstr, dummy_dst], axis=1)
    dstr = dstr2[:, :CH + 1]
    srci = jnp.stack([srcr, srcr + NPD], axis=0)

    degc = _deg_count(dstr2)
    deg = degc[0, :N, 0] + 1.0
    dis = jnp.concatenate([deg ** -0.5, jnp.zeros((NPD - N,), jnp.float32)])
    dis2d = dis[:, None]

    xp = jnp.concatenate([x, jnp.zeros((NPD - N, D), x.dtype)], axis=0)
    zt = _matmul_scaled(xp, dis2d, params['conv0_W'])
    gfmf = []
    counts = None
    for i in range(L):
        eacc = _edge_agg(zt.reshape(2 * NPD, 128), srci, dstr)
        wn = params[f'conv{i + 1}_W'] if i + 1 < L else params['conv0_W'][:H]
        zt, s2, smf, cnt = _mega(
            eacc, dis2d, batch2d, params[f'conv{i}_b'].reshape(1, H),
            params[f'norm{i}_w'].reshape(1, H),
            params[f'norm{i}_b'].reshape(1, H),
            params[f'norm{i}_ms'].reshape(1, H),
            params[f'mstnorm{i}_w'].reshape(1, H),
            params[f'mstnorm{i}_b'].reshape(1, H),
            params[f'mstnorm{i}_ms'].reshape(1, H), wn)
        counts = jnp.maximum(cnt, 1.0)
        gfmf.append(jnp.concatenate([s2 / counts, smf / counts], axis=1))

    out = _merge(jnp.stack(gfmf, 0), params['merge_W'],
                 params['merge_b'].reshape(1, H))
    return out
